# Initial kernel scaffold; baseline (speedup 1.0000x reference)
import jax
import jax.numpy as jnp
from jax.experimental import pallas as pl


def kernel(CLS_img, CLS_text, W1, b1, W2, b2, k):
    raise NotImplementedError("write your pallas kernel here")


# speedup vs baseline: 2.2527x; 2.2527x over previous
import functools
import math

import jax
import jax.numpy as jnp
from jax import lax
from jax.experimental import pallas as pl
from jax.experimental.pallas import tpu as pltpu
from jax.experimental.pallas import tpu_sc as plsc

B, N, D = 32, 8192, 768
H = D // 2
K = 256
TEMP = 0.1
BR = 2048
R = B * N


def _scores_body(a_ref, b_ref, w1_ref, b1_ref, w2_ref, b2_ref, out_ref):
    fused = (a_ref[...] + b_ref[...]) / 2.0
    h = jnp.dot(fused, w1_ref[...], preferred_element_type=jnp.float32)
    h = h + b1_ref[...]
    h = 0.5 * h * (1.0 + lax.erf(h / math.sqrt(2.0)))
    s = jnp.dot(h, w2_ref[...], preferred_element_type=jnp.float32)
    out_ref[...] = s + b2_ref[...]


def _scores(img2d, text2d, W1, b1, W2, b2):
    grid = (R // BR,)
    return pl.pallas_call(
        _scores_body,
        grid=grid,
        in_specs=[
            pl.BlockSpec((BR, D), lambda i: (i, 0)),
            pl.BlockSpec((BR, D), lambda i: (i, 0)),
            pl.BlockSpec((D, H), lambda i: (0, 0)),
            pl.BlockSpec((1, H), lambda i: (0, 0)),
            pl.BlockSpec((H, 1), lambda i: (0, 0)),
            pl.BlockSpec((1, 1), lambda i: (0, 0)),
        ],
        out_specs=pl.BlockSpec((BR, 1), lambda i: (i, 0)),
        out_shape=jax.ShapeDtypeStruct((R, 1), jnp.float32),
    )(img2d, text2d, W1, b1.reshape(1, H), W2, b2.reshape(1, 1))


def _topk_body(s_ref, probs_ref, idx_ref, fidx_ref, work_ref):
    x = s_ref[...] / TEMP
    m = jnp.max(x, axis=0, keepdims=True)
    e = jnp.exp(x - m)
    p = e / jnp.sum(e, axis=0, keepdims=True)
    probs_ref[...] = p
    work_ref[...] = p
    iota = lax.broadcasted_iota(jnp.int32, (N, B), 0)
    colbase = lax.broadcasted_iota(jnp.int32, (1, B), 1) * N

    def body(j, carry):
        w = work_ref[...]
        mx = jnp.max(w, axis=0, keepdims=True)
        cand = jnp.where(w == mx, iota, N)
        idx = jnp.min(cand, axis=0, keepdims=True)
        idx_ref[pl.ds(j, 1), :] = idx
        fidx_ref[pl.ds(j, 1), :] = idx + colbase
        work_ref[...] = jnp.where(iota == idx, -1.0, w)
        return carry

    lax.fori_loop(0, K, body, 0)


def _softmax_topk(scores_t):
    return pl.pallas_call(
        _topk_body,
        out_shape=(
            jax.ShapeDtypeStruct((N, B), jnp.float32),
            jax.ShapeDtypeStruct((K, B), jnp.int32),
            jax.ShapeDtypeStruct((K, B), jnp.int32),
        ),
        scratch_shapes=[pltpu.VMEM((N, B), jnp.float32)],
    )(scores_t)


_NC = 2
_NS = 16
_NW = _NC * _NS
_ROWS_W = (B * K) // _NW
_CH = 64
_NCH = _ROWS_W // _CH


def _sc_gather(img2d, text2d, flat_idx):
    mesh = plsc.VectorSubcoreMesh(core_axis_name="c", subcore_axis_name="s")

    @functools.partial(
        pl.kernel,
        mesh=mesh,
        out_type=jax.ShapeDtypeStruct((B * K, D), jnp.float32),
        scratch_types=[
            pltpu.VMEM((_CH,), jnp.int32),
            pltpu.VMEM((_CH, D), jnp.float32),
            pltpu.VMEM((_CH, D), jnp.float32),
            pltpu.SemaphoreType.DMA,
            pltpu.SemaphoreType.DMA,
        ],
    )
    def gather_kernel(img_hbm, text_hbm, idx_hbm, out_hbm, idx_v, a_v, b_v,
                      sem_a, sem_b):
        wid = lax.axis_index("s") * _NC + lax.axis_index("c")
        base = wid * _ROWS_W

        def chunk(ci, carry):
            cb = base + ci * _CH
            pltpu.sync_copy(idx_hbm.at[pl.ds(cb, _CH)], idx_v)
            cp_a = pltpu.async_copy(img_hbm.at[idx_v], a_v, sem_a)
            cp_b = pltpu.async_copy(text_hbm.at[idx_v], b_v, sem_b)
            cp_a.wait()
            cp_b.wait()

            def row(r, c2):
                for j in range(D // 16):
                    sl = pl.ds(j * 16, 16)
                    a_v[r, sl] = (a_v[r, sl] + b_v[r, sl]) / 2.0
                return c2

            lax.fori_loop(0, _CH, row, 0)
            pltpu.sync_copy(a_v, out_hbm.at[pl.ds(cb, _CH)])
            return carry

        lax.fori_loop(0, _NCH, chunk, 0)

    return gather_kernel(img2d, text2d, flat_idx)


def kernel(CLS_img, CLS_text, W1, b1, W2, b2, k):
    img2d = CLS_img.reshape(R, D)
    text2d = CLS_text.reshape(R, D)
    scores = _scores(img2d, text2d, W1, b1, W2, b2)
    scores_t = scores.reshape(B, N).T
    probs_t, idx_t, fidx_t = _softmax_topk(scores_t)
    importance_probs = probs_t.T
    topk_indices = idx_t.T
    flat_idx = fidx_t.T.reshape(B * K)
    selected = _sc_gather(img2d, text2d, flat_idx)
    selected_features = selected.reshape(B, K, D)
    return (selected_features, topk_indices, importance_probs)
